# baseline (speedup 1.0000x reference)
import functools

import jax
import jax.numpy as jnp
from jax import lax
from jax.experimental import pallas as pl
from jax.experimental.pallas import tpu as pltpu
from jax.experimental.pallas import tpu_sc as plsc

M, F, C, B, R = 10000, 3072, 100, 1024, 1024
CP = 128

NC, NS = 2, 16
NW = NC * NS
BPW = B // NW
RPW = R // NW
ROWS_BLK = 1000

_SC_PARAMS = pltpu.CompilerParams(needs_layout_passes=False)
_SC_MESH = plsc.VectorSubcoreMesh(core_axis_name="c", subcore_axis_name="s")


def _copy_body(img_in, logits_in, label_in, img_out, logits_out, label_out):
    img_out[...] = img_in[...]
    logits_out[:, :C] = logits_in[...]
    logits_out[:, C:] = jnp.zeros((ROWS_BLK, CP - C), jnp.float32)
    label_out[...] = label_in[...]


def _copy3(mem_img, mem_logits, mem_label2d):
    grid = (M // ROWS_BLK,)
    return pl.pallas_call(
        _copy_body,
        grid=grid,
        in_specs=[
            pl.BlockSpec((ROWS_BLK, F), lambda i: (i, 0)),
            pl.BlockSpec((ROWS_BLK, C), lambda i: (i, 0)),
            pl.BlockSpec((ROWS_BLK, 1), lambda i: (i, 0)),
        ],
        out_specs=[
            pl.BlockSpec((ROWS_BLK, F), lambda i: (i, 0)),
            pl.BlockSpec((ROWS_BLK, CP), lambda i: (i, 0)),
            pl.BlockSpec((ROWS_BLK, 1), lambda i: (i, 0)),
        ],
        out_shape=[
            jax.ShapeDtypeStruct((M, F), jnp.float32),
            jax.ShapeDtypeStruct((M, CP), jnp.float32),
            jax.ShapeDtypeStruct((M, 1), jnp.int32),
        ],
    )(mem_img, mem_logits, mem_label2d)


def _sc_body(img_ref, logits_ref, label_ref,
             x_hbm, xl_hbm, y_hbm, idx_hbm, ridx_hbm,
             rx_hbm, rl_hbm, ry_hbm,
             idx_v, y_v, winner_v, tsel_v, wsel_v, ysel_v, ri_v, rlab_v,
             rows_v, lrow_v, sem0, sem1, sem2, sem3):
    wid = lax.axis_index("s") * NC + lax.axis_index("c")
    base = wid * BPW
    lanes = lax.iota(jnp.int32, 16)
    zero16 = jnp.zeros((16,), jnp.int32)

    cp_idx = pltpu.async_copy(idx_hbm, idx_v, sem0)
    cp_y = pltpu.async_copy(y_hbm, y_v, sem1)
    cp_ri = pltpu.async_copy(ridx_hbm.at[pl.ds(base, RPW)], ri_v, sem2)
    cp_idx.wait()
    cp_y.wait()
    cp_ri.wait()

    for k in range(RPW // 16):
        rk = ri_v[pl.ds(k * 16, 16)]
        plsc.store_scatter(winner_v, [rk], zero16)

    def _build(c, _):
        tvec = idx_v[pl.ds(c * 16, 16)]
        for k in range(16):
            plsc.store_scatter(winner_v, [tvec], zero16 + (c * 16 + k + 1),
                               mask=lanes == k)
        return 0
    lax.fori_loop(0, B // 16, _build, 0)

    cp_lab = pltpu.async_copy(label_ref.at[ri_v], rlab_v, sem3)

    pwins = []
    for k in range(BPW // 16):
        tk = idx_v[pl.ds(base + k * 16, 16)]
        wk = plsc.load_gather(winner_v, [tk]) - 1
        tsel_v[pl.ds(k * 16, 16)] = tk
        wsel_v[pl.ds(k * 16, 16)] = wk
        ysel_v[pl.ds(k * 16, 16)] = plsc.load_gather(y_v, [wk])
        rk = ri_v[pl.ds(k * 16, 16)]
        pwins.append(plsc.load_gather(winner_v, [rk]) - 1)

    g0 = pltpu.async_copy(x_hbm.at[wsel_v], rows_v, sem0)
    g1 = pltpu.async_copy(xl_hbm.at[wsel_v], lrow_v, sem1)
    g0.wait()
    g1.wait()
    cs0 = pltpu.async_copy(rows_v, img_ref.at[tsel_v], sem0)
    cs1 = pltpu.async_copy(lrow_v, logits_ref.at[tsel_v], sem1)
    cs2 = pltpu.async_copy(ysel_v, label_ref.at[tsel_v], sem2)
    cs0.wait()
    cs1.wait()

    r0 = pltpu.async_copy(img_ref.at[ri_v], rows_v, sem0)
    r1 = pltpu.async_copy(logits_ref.at[ri_v], lrow_v, sem1)
    cp_lab.wait()
    for k in range(RPW // 16):
        wk = pwins[k]
        ylk = plsc.load_gather(y_v, [jnp.maximum(wk, 0)])
        cur = rlab_v[pl.ds(k * 16, 16)]
        rlab_v[pl.ds(k * 16, 16)] = jnp.where(wk >= 0, ylk, cur)
    r0.wait()
    r1.wait()

    for k in range(RPW // 16):
        wk = pwins[k]
        for lane in range(16):
            win = wk[lane]
            j = k * 16 + lane

            @pl.when(win >= 0)
            def _(win=win, j=j):
                pltpu.sync_copy(x_hbm.at[pl.ds(win, 1)],
                                rows_v.at[pl.ds(j, 1)])
                pltpu.sync_copy(xl_hbm.at[pl.ds(win, 1)],
                                lrow_v.at[pl.ds(j, 1)])

    w0 = pltpu.async_copy(rows_v, rx_hbm.at[pl.ds(base, RPW)], sem0)
    w1 = pltpu.async_copy(lrow_v, rl_hbm.at[pl.ds(base, RPW)], sem1)
    w2 = pltpu.async_copy(rlab_v, ry_hbm.at[pl.ds(base, RPW)], sem3)
    w0.wait()
    w1.wait()
    w2.wait()
    cs2.wait()


_sc_call = functools.partial(
    pl.kernel,
    out_type=(
        jax.ShapeDtypeStruct((R, F), jnp.float32),
        jax.ShapeDtypeStruct((R, CP), jnp.float32),
        jax.ShapeDtypeStruct((R,), jnp.int32),
    ),
    mesh=_SC_MESH,
    compiler_params=_SC_PARAMS,
    scratch_types=[
        pltpu.VMEM((B,), jnp.int32),
        pltpu.VMEM((B,), jnp.int32),
        pltpu.VMEM((M,), jnp.int32),
        pltpu.VMEM((BPW,), jnp.int32),
        pltpu.VMEM((BPW,), jnp.int32),
        pltpu.VMEM((BPW,), jnp.int32),
        pltpu.VMEM((RPW,), jnp.int32),
        pltpu.VMEM((RPW,), jnp.int32),
        pltpu.VMEM((BPW, F), jnp.float32),
        pltpu.VMEM((BPW, CP), jnp.float32),
        pltpu.SemaphoreType.DMA,
        pltpu.SemaphoreType.DMA,
        pltpu.SemaphoreType.DMA,
        pltpu.SemaphoreType.DMA,
    ],
)(_sc_body)


def kernel(mem_img, mem_logits, mem_label, x, logits, y, idx, retrieve_idx):
    logits_p = jnp.pad(logits, ((0, 0), (0, CP - C)))
    img_c, logits_c, label_c = _copy3(mem_img, mem_logits,
                                      mem_label.reshape(M, 1))
    img_r = jax.new_ref(img_c)
    logits_r = jax.new_ref(logits_c)
    label_r = jax.new_ref(label_c.reshape(M))
    r_x, r_l, r_y = _sc_call(img_r, logits_r, label_r,
                             x, logits_p, y, idx, retrieve_idx)
    return (jax.freeze(img_r), jax.freeze(logits_r)[:, :C],
            jax.freeze(label_r), r_x, r_l[:, :C], r_y)


# speedup vs baseline: 1.0078x; 1.0078x over previous
import functools

import jax
import jax.numpy as jnp
from jax import lax
from jax.experimental import pallas as pl
from jax.experimental.pallas import tpu as pltpu
from jax.experimental.pallas import tpu_sc as plsc

M, F, C, B, R = 10000, 3072, 100, 1024, 1024
CP = 128

NC, NS = 2, 16
NW = NC * NS
BPW = B // NW
RPW = R // NW
ROWS_BLK = 1000

_SC_PARAMS = pltpu.CompilerParams(needs_layout_passes=False)
_SC_MESH = plsc.VectorSubcoreMesh(core_axis_name="c", subcore_axis_name="s")


def _copy_body(img_in, logits_in, label_in, img_out, logits_out, label_out):
    img_out[...] = img_in[...]
    logits_out[:, :C] = logits_in[...]
    logits_out[:, C:] = jnp.zeros((ROWS_BLK, CP - C), jnp.float32)
    label_out[...] = label_in[...]


def _copy3(mem_img, mem_logits, mem_label2d):
    grid = (M // ROWS_BLK,)
    return pl.pallas_call(
        _copy_body,
        grid=grid,
        in_specs=[
            pl.BlockSpec((ROWS_BLK, F), lambda i: (i, 0)),
            pl.BlockSpec((ROWS_BLK, C), lambda i: (i, 0)),
            pl.BlockSpec((ROWS_BLK, 1), lambda i: (i, 0)),
        ],
        out_specs=[
            pl.BlockSpec((ROWS_BLK, F), lambda i: (i, 0)),
            pl.BlockSpec((ROWS_BLK, CP), lambda i: (i, 0)),
            pl.BlockSpec((ROWS_BLK, 1), lambda i: (i, 0)),
        ],
        out_shape=[
            jax.ShapeDtypeStruct((M, F), jnp.float32),
            jax.ShapeDtypeStruct((M, CP), jnp.float32),
            jax.ShapeDtypeStruct((M, 1), jnp.int32),
        ],
    )(mem_img, mem_logits, mem_label2d)


def _sc_body(img_ref, logits_ref, label_ref,
             x_hbm, xl_hbm, y_hbm, idx_hbm, ridx_hbm,
             rx_hbm, rl_hbm, ry_hbm,
             idx_v, y_v, winner_v, tsel_v, ysel_v, ri_v, rlab_v,
             tsel1_v, tsel2_v, wsel1_v, wsel2_v, ri1_v, ri2_v,
             ub_v, rb_v, ulb_v, rlb_v, sem0, sem1, sem2, sem3):
    wid = lax.axis_index("s") * NC + lax.axis_index("c")
    base = wid * BPW
    lanes = lax.iota(jnp.int32, 16)
    zero16 = jnp.zeros((16,), jnp.int32)

    cp_idx = pltpu.async_copy(idx_hbm, idx_v, sem0)
    cp_y = pltpu.async_copy(y_hbm, y_v, sem1)
    cp_ri = pltpu.async_copy(ridx_hbm.at[pl.ds(base, RPW)], ri_v, sem2)
    cp_idx.wait()
    cp_y.wait()
    cp_ri.wait()

    for k in range(RPW // 16):
        rk = ri_v[pl.ds(k * 16, 16)]
        plsc.store_scatter(winner_v, [rk], zero16)

    def _build(c, _):
        tvec = idx_v[pl.ds(c * 16, 16)]
        for k in range(16):
            plsc.store_scatter(winner_v, [tvec], zero16 + (c * 16 + k + 1),
                               mask=lanes == k)
        return 0
    lax.fori_loop(0, B // 16, _build, 0)

    cp_lab = pltpu.async_copy(label_ref.at[ri_v], rlab_v, sem3)

    pwins = []
    for k, (tselk, wselk, rik) in enumerate(((tsel1_v, wsel1_v, ri1_v),
                                             (tsel2_v, wsel2_v, ri2_v))):
        tk = idx_v[pl.ds(base + k * 16, 16)]
        wk = plsc.load_gather(winner_v, [tk]) - 1
        tsel_v[pl.ds(k * 16, 16)] = tk
        tselk[...] = tk
        wselk[...] = wk
        ysel_v[pl.ds(k * 16, 16)] = plsc.load_gather(y_v, [wk])
        rik[...] = ri_v[pl.ds(k * 16, 16)]
        pwins.append(plsc.load_gather(winner_v, [ri_v[pl.ds(k * 16, 16)]]) - 1)

    def _patch(k, rbuf, lbuf):
        wk = pwins[k]
        for lane in range(16):
            win = wk[lane]

            @pl.when(win >= 0)
            def _(win=win, lane=lane):
                pltpu.sync_copy(x_hbm.at[pl.ds(win, 1)],
                                rbuf.at[pl.ds(lane, 1)])
                pltpu.sync_copy(xl_hbm.at[pl.ds(win, 1)],
                                lbuf.at[pl.ds(lane, 1)])

    rg1 = pltpu.async_copy(img_ref.at[ri1_v], rb_v, sem2)
    rgl1 = pltpu.async_copy(logits_ref.at[ri1_v], rlb_v, sem3)
    ug1 = pltpu.async_copy(x_hbm.at[wsel1_v], ub_v, sem0)
    ugl1 = pltpu.async_copy(xl_hbm.at[wsel1_v], ulb_v, sem1)
    cs2 = pltpu.async_copy(ysel_v, label_ref.at[tsel_v], sem2)

    ug1.wait()
    ugl1.wait()
    us1 = pltpu.async_copy(ub_v, img_ref.at[tsel1_v], sem0)
    usl1 = pltpu.async_copy(ulb_v, logits_ref.at[tsel1_v], sem1)
    us1.wait()
    usl1.wait()
    ug2 = pltpu.async_copy(x_hbm.at[wsel2_v], ub_v, sem0)
    ugl2 = pltpu.async_copy(xl_hbm.at[wsel2_v], ulb_v, sem1)

    cp_lab.wait()
    for k in range(2):
        wk = pwins[k]
        ylk = plsc.load_gather(y_v, [jnp.maximum(wk, 0)])
        cur = rlab_v[pl.ds(k * 16, 16)]
        rlab_v[pl.ds(k * 16, 16)] = jnp.where(wk >= 0, ylk, cur)
    rg1.wait()
    rgl1.wait()
    _patch(0, rb_v, rlb_v)
    w1 = pltpu.async_copy(rb_v, rx_hbm.at[pl.ds(base, 16)], sem2)
    wl1 = pltpu.async_copy(rlb_v, rl_hbm.at[pl.ds(base, 16)], sem3)

    ug2.wait()
    ugl2.wait()
    us2 = pltpu.async_copy(ub_v, img_ref.at[tsel2_v], sem0)
    usl2 = pltpu.async_copy(ulb_v, logits_ref.at[tsel2_v], sem1)

    w1.wait()
    wl1.wait()
    rg2 = pltpu.async_copy(img_ref.at[ri2_v], rb_v, sem2)
    rgl2 = pltpu.async_copy(logits_ref.at[ri2_v], rlb_v, sem3)
    rg2.wait()
    rgl2.wait()
    _patch(1, rb_v, rlb_v)
    w2 = pltpu.async_copy(rb_v, rx_hbm.at[pl.ds(base + 16, 16)], sem2)
    wl2 = pltpu.async_copy(rlb_v, rl_hbm.at[pl.ds(base + 16, 16)], sem3)
    wy = pltpu.async_copy(rlab_v, ry_hbm.at[pl.ds(base, RPW)], sem3)

    us2.wait()
    usl2.wait()
    w2.wait()
    wl2.wait()
    wy.wait()
    cs2.wait()


_sc_call = functools.partial(
    pl.kernel,
    out_type=(
        jax.ShapeDtypeStruct((R, F), jnp.float32),
        jax.ShapeDtypeStruct((R, CP), jnp.float32),
        jax.ShapeDtypeStruct((R,), jnp.int32),
    ),
    mesh=_SC_MESH,
    compiler_params=_SC_PARAMS,
    scratch_types=[
        pltpu.VMEM((B,), jnp.int32),
        pltpu.VMEM((B,), jnp.int32),
        pltpu.VMEM((M,), jnp.int32),
        pltpu.VMEM((BPW,), jnp.int32),
        pltpu.VMEM((BPW,), jnp.int32),
        pltpu.VMEM((RPW,), jnp.int32),
        pltpu.VMEM((RPW,), jnp.int32),
        pltpu.VMEM((16,), jnp.int32),
        pltpu.VMEM((16,), jnp.int32),
        pltpu.VMEM((16,), jnp.int32),
        pltpu.VMEM((16,), jnp.int32),
        pltpu.VMEM((16,), jnp.int32),
        pltpu.VMEM((16,), jnp.int32),
        pltpu.VMEM((16, F), jnp.float32),
        pltpu.VMEM((16, F), jnp.float32),
        pltpu.VMEM((16, CP), jnp.float32),
        pltpu.VMEM((16, CP), jnp.float32),
        pltpu.SemaphoreType.DMA,
        pltpu.SemaphoreType.DMA,
        pltpu.SemaphoreType.DMA,
        pltpu.SemaphoreType.DMA,
    ],
)(_sc_body)


def kernel(mem_img, mem_logits, mem_label, x, logits, y, idx, retrieve_idx):
    logits_p = jnp.pad(logits, ((0, 0), (0, CP - C)))
    img_c, logits_c, label_c = _copy3(mem_img, mem_logits,
                                      mem_label.reshape(M, 1))
    img_r = jax.new_ref(img_c)
    logits_r = jax.new_ref(logits_c)
    label_r = jax.new_ref(label_c.reshape(M))
    r_x, r_l, r_y = _sc_call(img_r, logits_r, label_r,
                             x, logits_p, y, idx, retrieve_idx)
    return (jax.freeze(img_r), jax.freeze(logits_r)[:, :C],
            jax.freeze(label_r), r_x, r_l[:, :C], r_y)
